# baseline (speedup 1.0000x reference)
import jax
import jax.numpy as jnp
from jax.experimental import pallas as pl

_LAYER_SIZES = [256, 256]


def _one_layer(key, v_indices, output_size, features, adj, w1, w2):
    support = adj[v_indices, :]
    col_sum = jnp.sum(support, axis=0)
    nei_mask = col_sum != 0
    inv_num_neis = (1.0 / jnp.sum(nei_mask).astype(jnp.float64)).astype(jnp.float32)
    h_v = features[v_indices]
    h_u = features
    attention = jnp.matmul(h_v, w1) + jnp.matmul(h_u, w2).reshape(1, -1) + 1.0
    attention = inv_num_neis * jax.nn.relu(attention)
    p1 = jnp.sum(support * attention, axis=0)
    p = p1 / jnp.sum(p1)
    sampled = jax.random.choice(key, adj.shape[1], shape=(output_size,), replace=True, p=p)
    u_sampled = sampled
    support_s = support[:, sampled]
    sampled_p1 = p1[sampled]
    t_diag = jnp.diag(1.0 / (sampled_p1 * output_size))
    support_s = jnp.matmul(support_s, t_diag)
    return u_sampled, support_s, (nei_mask, p1 / jnp.sum(p1))


def kernel(features, adj, w1, w2, v):
    key = jax.random.key(42)
    num_layers = len(_LAYER_SIZES)
    all_support = [None] * num_layers
    all_p_u = [None] * num_layers
    cur = v
    for i in range(num_layers - 1, -1, -1):
        u_sampled, sup, var_need = _one_layer(
            jax.random.fold_in(key, i), cur, _LAYER_SIZES[i], features, adj, w1, w2
        )
        all_support[i] = sup
        all_p_u[i] = var_need
        cur = u_sampled
    u_mask, p_u = all_p_u[-1]
    p_u = p_u.reshape(-1, 1)
    m = u_mask.astype(features.dtype).reshape(-1, 1)
    means = jnp.sum(features * m, axis=0)
    feat = features - means
    loss = jnp.mean(jnp.sum(feat * feat * p_u, axis=0))
    sampled_X0 = features[cur]
    return (sampled_X0, all_support[0], all_support[1], loss)
